# Initial kernel scaffold; baseline (speedup 1.0000x reference)
import jax
import jax.numpy as jnp
from jax.experimental import pallas as pl


def kernel(x, b_wh, W_wh, Wk, Wq, Wv, W_proj, b_proj, w_gate, W1, b1, W2, b2):
    raise NotImplementedError("write your pallas kernel here")


# speedup vs baseline: 1.5496x; 1.5496x over previous
import functools

import jax
import jax.numpy as jnp
from jax.experimental import pallas as pl
from jax.experimental.pallas import tpu as pltpu

_TOPK = 4


def _mm(a, b, ca, cb):
    return jax.lax.dot_general(
        a, b, (((ca,), (cb,)), ((), ())), preferred_element_type=jnp.float32
    )


def _kern(x_ref, bwh_ref, Wwh_ref, Wk_ref, Wq_ref, Wv_ref, Wp_ref, bp_ref,
          wg_ref, W1_ref, b1_ref, W2_ref, b2_ref, out_ref, *, heads, scale):
    T = x_ref.shape[1]
    E = wg_ref.shape[1]

    x = x_ref[0] - bwh_ref[...]
    h = _mm(x, Wwh_ref[...], 1, 1)

    ri = jax.lax.broadcasted_iota(jnp.int32, (T, T), 0)
    ci = jax.lax.broadcasted_iota(jnp.int32, (T, T), 1)
    causal = ci <= ri
    att_heads = []
    for hh in range(heads):
        q = _mm(h, Wq_ref[hh], 1, 1)
        k = _mm(h, Wk_ref[hh], 1, 1)
        v = _mm(h, Wv_ref[hh], 1, 1)
        wei = _mm(q, k, 1, 1) * scale
        wei = jnp.where(causal, wei, -1e30)
        wei = jnp.exp(wei - jnp.max(wei, axis=1, keepdims=True))
        wei = wei / jnp.sum(wei, axis=1, keepdims=True)
        att_heads.append(_mm(wei, v, 1, 0))
    att = jnp.concatenate(att_heads, axis=1)
    h2 = _mm(att, Wp_ref[...], 1, 1) + bp_ref[...]

    m = _mm(h2, wg_ref[...], 1, 0)
    jidx = jax.lax.broadcasted_iota(jnp.int32, (T, E), 1)
    keeps = []
    for e in range(E):
        me = m[:, e:e + 1]
        beats = ((m > me) | ((m == me) & (jidx < e))).astype(jnp.float32)
        rank = jnp.sum(beats, axis=1, keepdims=True)
        keeps.append(jnp.where(rank < _TOPK, 1.0, 0.0))
    keep = jnp.concatenate(keeps, axis=1)
    masked = jnp.where(keep > 0.5, m, -1e30)
    ex = jnp.exp(masked - jnp.max(masked, axis=1, keepdims=True))
    ex = ex * keep
    g = ex / jnp.sum(ex, axis=1, keepdims=True)

    b1 = b1_ref[...]
    b2 = b2_ref[...]
    S1 = jnp.zeros_like(h2)
    S2 = jnp.zeros_like(h2)
    for e in range(E):
        t = jnp.maximum(_mm(h2, W1_ref[e], 1, 1) + b1[e:e + 1, :], 0.0)
        o = _mm(t, W2_ref[e], 1, 1) + b2[e:e + 1, :]
        S2 = S2 + o
        S1 = S1 + g[:, e:e + 1] * o
    out_ref[0] = S1 * S2


@jax.jit
def kernel(x, b_wh, W_wh, Wk, Wq, Wv, W_proj, b_proj, w_gate, W1, b1, W2, b2):
    B, T, D_IN = x.shape
    D = W_wh.shape[0]
    H, HS, _ = Wq.shape
    E = w_gate.shape[1]
    D2 = W1.shape[1]

    full = lambda shape: pl.BlockSpec(shape, lambda b: (0,) * len(shape))
    out = pl.pallas_call(
        functools.partial(_kern, heads=H, scale=D ** -0.5),
        grid=(B,),
        in_specs=[
            pl.BlockSpec((1, T, D_IN), lambda b: (b, 0, 0)),
            full((1, D_IN)),
            full((D, D_IN)),
            full((H, HS, D)),
            full((H, HS, D)),
            full((H, HS, D)),
            full((D, H * HS)),
            full((1, D)),
            full((D, E)),
            full((E, D2, D)),
            full((E, D2)),
            full((E, D, D2)),
            full((E, D)),
        ],
        out_specs=pl.BlockSpec((1, T, D), lambda b: (b, 0, 0)),
        out_shape=jax.ShapeDtypeStruct((B, T, D), jnp.float32),
        compiler_params=pltpu.CompilerParams(
            dimension_semantics=("parallel",),
        ),
    )(x, b_wh.reshape(1, D_IN), W_wh, Wk, Wq, Wv, W_proj,
      b_proj.reshape(1, D), w_gate, W1, b1, W2, b2)
    return out


# speedup vs baseline: 1.5653x; 1.0101x over previous
import functools

import jax
import jax.numpy as jnp
from jax.experimental import pallas as pl
from jax.experimental.pallas import tpu as pltpu

_TOPK = 4


def _mm(a, b, ca, cb):
    return jax.lax.dot_general(
        a, b, (((ca,), (cb,)), ((), ())), preferred_element_type=jnp.float32
    )


def _kern(x_ref, bwh_ref, Wwh_ref, Wk_ref, Wq_ref, Wv_ref, Wp_ref, bp_ref,
          wg_ref, W1_ref, b1_ref, W2_ref, b2_ref, out_ref, *, heads, scale):
    P, T = x_ref.shape[0], x_ref.shape[1]
    E = wg_ref.shape[1]

    ri = jax.lax.broadcasted_iota(jnp.int32, (T, T), 0)
    ci = jax.lax.broadcasted_iota(jnp.int32, (T, T), 1)
    causal = ci <= ri

    h2_rows = []
    for p in range(P):
        x = x_ref[p] - bwh_ref[...]
        h = _mm(x, Wwh_ref[...], 1, 1)
        att_heads = []
        for hh in range(heads):
            q = _mm(h, Wq_ref[hh], 1, 1)
            k = _mm(h, Wk_ref[hh], 1, 1)
            v = _mm(h, Wv_ref[hh], 1, 1)
            wei = _mm(q, k, 1, 1) * scale
            wei = jnp.where(causal, wei, -1e30)
            wei = jnp.exp(wei - jnp.max(wei, axis=1, keepdims=True))
            wei = wei / jnp.sum(wei, axis=1, keepdims=True)
            att_heads.append(_mm(wei, v, 1, 0))
        att = jnp.concatenate(att_heads, axis=1)
        h2_rows.append(_mm(att, Wp_ref[...], 1, 1) + bp_ref[...])
    h2 = jnp.concatenate(h2_rows, axis=0)

    m = _mm(h2, wg_ref[...], 1, 0)
    jidx = jax.lax.broadcasted_iota(jnp.int32, (P * T, E), 1)
    keeps = []
    for e in range(E):
        me = m[:, e:e + 1]
        beats = ((m > me) | ((m == me) & (jidx < e))).astype(jnp.float32)
        rank = jnp.sum(beats, axis=1, keepdims=True)
        keeps.append(jnp.where(rank < _TOPK, 1.0, 0.0))
    keep = jnp.concatenate(keeps, axis=1)
    masked = jnp.where(keep > 0.5, m, -1e30)
    ex = jnp.exp(masked - jnp.max(masked, axis=1, keepdims=True))
    ex = ex * keep
    g = ex / jnp.sum(ex, axis=1, keepdims=True)

    b1 = b1_ref[...]
    b2 = b2_ref[...]
    S1 = jnp.zeros_like(h2)
    S2 = jnp.zeros_like(h2)
    for e in range(E):
        t = jnp.maximum(_mm(h2, W1_ref[e], 1, 1) + b1[e:e + 1, :], 0.0)
        o = _mm(t, W2_ref[e], 1, 1) + b2[e:e + 1, :]
        S2 = S2 + o
        S1 = S1 + g[:, e:e + 1] * o
    out = S1 * S2
    for p in range(P):
        out_ref[p] = out[p * T:(p + 1) * T]


@jax.jit
def kernel(x, b_wh, W_wh, Wk, Wq, Wv, W_proj, b_proj, w_gate, W1, b1, W2, b2):
    B, T, D_IN = x.shape
    D = W_wh.shape[0]
    H, HS, _ = Wq.shape
    E = w_gate.shape[1]
    D2 = W1.shape[1]

    P = 4 if B % 4 == 0 else 1
    full = lambda shape: pl.BlockSpec(shape, lambda b: (0,) * len(shape))
    out = pl.pallas_call(
        functools.partial(_kern, heads=H, scale=D ** -0.5),
        grid=(B // P,),
        in_specs=[
            pl.BlockSpec((P, T, D_IN), lambda b: (b, 0, 0)),
            full((1, D_IN)),
            full((D, D_IN)),
            full((H, HS, D)),
            full((H, HS, D)),
            full((H, HS, D)),
            full((D, H * HS)),
            full((1, D)),
            full((D, E)),
            full((E, D2, D)),
            full((E, D2)),
            full((E, D, D2)),
            full((E, D)),
        ],
        out_specs=pl.BlockSpec((P, T, D), lambda b: (b, 0, 0)),
        out_shape=jax.ShapeDtypeStruct((B, T, D), jnp.float32),
        compiler_params=pltpu.CompilerParams(
            dimension_semantics=("parallel",),
        ),
    )(x, b_wh.reshape(1, D_IN), W_wh, Wk, Wq, Wv, W_proj,
      b_proj.reshape(1, D), w_gate, W1, b1, W2, b2)
    return out


# speedup vs baseline: 1.6146x; 1.0315x over previous
import functools

import jax
import jax.numpy as jnp
from jax.experimental import pallas as pl
from jax.experimental.pallas import tpu as pltpu

_TOPK = 4


def _mm(a, b, ca, cb):
    return jax.lax.dot_general(
        a, b, (((ca,), (cb,)), ((), ())), preferred_element_type=jnp.float32
    )


def _kern(x_ref, bwh_ref, Wwh_ref, Wk_ref, Wq_ref, Wv_ref, Wp_ref, bp_ref,
          wg_ref, W1_ref, b1_ref, W2_ref, b2_ref, out_ref, *, heads, scale):
    P, T = x_ref.shape[0], x_ref.shape[1]
    E = wg_ref.shape[1]

    ri = jax.lax.broadcasted_iota(jnp.int32, (T, T), 0)
    ci = jax.lax.broadcasted_iota(jnp.int32, (T, T), 1)
    causal = ci <= ri

    h2_rows = []
    for p in range(P):
        x = x_ref[p] - bwh_ref[...]
        h = _mm(x, Wwh_ref[...], 1, 1)
        att_heads = []
        for hh in range(heads):
            q = _mm(h, Wq_ref[hh], 1, 1)
            k = _mm(h, Wk_ref[hh], 1, 1)
            v = _mm(h, Wv_ref[hh], 1, 1)
            wei = _mm(q, k, 1, 1) * scale
            wei = jnp.where(causal, wei, -1e30)
            wei = jnp.exp(wei - jnp.max(wei, axis=1, keepdims=True))
            wei = wei / jnp.sum(wei, axis=1, keepdims=True)
            att_heads.append(_mm(wei, v, 1, 0))
        att = jnp.concatenate(att_heads, axis=1)
        h2_rows.append(_mm(att, Wp_ref[...], 1, 1) + bp_ref[...])
    h2 = jnp.concatenate(h2_rows, axis=0)

    m = _mm(h2, wg_ref[...], 1, 0)
    jidx = jax.lax.broadcasted_iota(jnp.int32, (P * T, E), 1)
    keeps = []
    for e in range(E):
        me = m[:, e:e + 1]
        beats = ((m > me) | ((m == me) & (jidx < e))).astype(jnp.float32)
        rank = jnp.sum(beats, axis=1, keepdims=True)
        keeps.append(jnp.where(rank < _TOPK, 1.0, 0.0))
    keep = jnp.concatenate(keeps, axis=1)
    masked = jnp.where(keep > 0.5, m, -1e30)
    ex = jnp.exp(masked - jnp.max(masked, axis=1, keepdims=True))
    ex = ex * keep
    g = ex / jnp.sum(ex, axis=1, keepdims=True)

    b1 = b1_ref[...]
    b2 = b2_ref[...]
    S1 = jnp.zeros_like(h2)
    S2 = jnp.zeros_like(h2)
    h2b = h2.astype(jnp.bfloat16)
    for e in range(E):
        t = jnp.maximum(_mm(h2b, W1_ref[e].astype(jnp.bfloat16), 1, 1)
                        + b1[e:e + 1, :], 0.0)
        o = _mm(t.astype(jnp.bfloat16), W2_ref[e].astype(jnp.bfloat16), 1, 1
                ) + b2[e:e + 1, :]
        S2 = S2 + o
        S1 = S1 + g[:, e:e + 1] * o
    out = S1 * S2
    for p in range(P):
        out_ref[p] = out[p * T:(p + 1) * T]


@jax.jit
def kernel(x, b_wh, W_wh, Wk, Wq, Wv, W_proj, b_proj, w_gate, W1, b1, W2, b2):
    B, T, D_IN = x.shape
    D = W_wh.shape[0]
    H, HS, _ = Wq.shape
    E = w_gate.shape[1]
    D2 = W1.shape[1]

    P = 4 if B % 4 == 0 else 1
    full = lambda shape: pl.BlockSpec(shape, lambda b: (0,) * len(shape))
    out = pl.pallas_call(
        functools.partial(_kern, heads=H, scale=D ** -0.5),
        grid=(B // P,),
        in_specs=[
            pl.BlockSpec((P, T, D_IN), lambda b: (b, 0, 0)),
            full((1, D_IN)),
            full((D, D_IN)),
            full((H, HS, D)),
            full((H, HS, D)),
            full((H, HS, D)),
            full((D, H * HS)),
            full((1, D)),
            full((D, E)),
            full((E, D2, D)),
            full((E, D2)),
            full((E, D, D2)),
            full((E, D)),
        ],
        out_specs=pl.BlockSpec((P, T, D), lambda b: (b, 0, 0)),
        out_shape=jax.ShapeDtypeStruct((B, T, D), jnp.float32),
        compiler_params=pltpu.CompilerParams(
            dimension_semantics=("parallel",),
        ),
    )(x, b_wh.reshape(1, D_IN), W_wh, Wk, Wq, Wv, W_proj,
      b_proj.reshape(1, D), w_gate, W1, b1, W2, b2)
    return out


# speedup vs baseline: 1.8567x; 1.1500x over previous
import functools

import jax
import jax.numpy as jnp
from jax.experimental import pallas as pl
from jax.experimental.pallas import tpu as pltpu

_TOPK = 4


def _mm(a, b, ca, cb):
    return jax.lax.dot_general(
        a.astype(jnp.bfloat16), b.astype(jnp.bfloat16),
        (((ca,), (cb,)), ((), ())), preferred_element_type=jnp.float32
    )


def _kern(x_ref, bwh_ref, Wwh_ref, Wqkv_ref, Wp_ref, bp_ref,
          wg_ref, W1_ref, b1_ref, W2_ref, b2_ref, out_ref, *, heads, scale):
    P, T = x_ref.shape[0], x_ref.shape[1]
    E = wg_ref.shape[1]
    HS = Wqkv_ref.shape[0] // (3 * heads)

    ri = jax.lax.broadcasted_iota(jnp.int32, (T, T), 0)
    ci = jax.lax.broadcasted_iota(jnp.int32, (T, T), 1)
    causal = ci <= ri

    h2_rows = []
    for p in range(P):
        x = x_ref[p] - bwh_ref[...]
        h = _mm(x, Wwh_ref[...], 1, 1)
        qkv = _mm(h, Wqkv_ref[...], 1, 1)
        att_heads = []
        for hh in range(heads):
            q = qkv[:, hh * HS:(hh + 1) * HS]
            k = qkv[:, (heads + hh) * HS:(heads + hh + 1) * HS]
            v = qkv[:, (2 * heads + hh) * HS:(2 * heads + hh + 1) * HS]
            wei = _mm(q, k, 1, 1) * scale
            wei = jnp.exp(jnp.where(causal, wei, -1e30))
            wei = wei * (1.0 / jnp.sum(wei, axis=1, keepdims=True))
            att_heads.append(_mm(wei, v, 1, 0))
        att = jnp.concatenate(att_heads, axis=1)
        h2_rows.append(_mm(att, Wp_ref[...], 1, 1) + bp_ref[...])
    h2 = jnp.concatenate(h2_rows, axis=0)

    m = _mm(h2, wg_ref[...], 1, 0)
    jidx = jax.lax.broadcasted_iota(jnp.int32, (P * T, E), 1)
    keeps = []
    for e in range(E):
        me = m[:, e:e + 1]
        beats = ((m > me) | ((m == me) & (jidx < e))).astype(jnp.float32)
        rank = jnp.sum(beats, axis=1, keepdims=True)
        keeps.append(jnp.where(rank < _TOPK, 1.0, 0.0))
    keep = jnp.concatenate(keeps, axis=1)
    masked = jnp.where(keep > 0.5, m, -1e30)
    ex = jnp.exp(masked - jnp.max(masked, axis=1, keepdims=True))
    ex = ex * keep
    g = ex * (1.0 / jnp.sum(ex, axis=1, keepdims=True))

    b1 = b1_ref[...]
    b2 = b2_ref[...]
    S1 = jnp.zeros_like(h2)
    S2 = jnp.zeros_like(h2)
    h2b = h2.astype(jnp.bfloat16)
    for e in range(E):
        t = jnp.maximum(_mm(h2b, W1_ref[e], 1, 1) + b1[e:e + 1, :], 0.0)
        o = _mm(t, W2_ref[e], 1, 1) + b2[e:e + 1, :]
        S2 = S2 + o
        S1 = S1 + g[:, e:e + 1] * o
    out = S1 * S2
    out_ref[...] = out.reshape(out_ref.shape)


@jax.jit
def kernel(x, b_wh, W_wh, Wk, Wq, Wv, W_proj, b_proj, w_gate, W1, b1, W2, b2):
    B, T, D_IN = x.shape
    D = W_wh.shape[0]
    H, HS, _ = Wq.shape
    E = w_gate.shape[1]
    D2 = W1.shape[1]

    Wqkv = jnp.concatenate(
        [Wq.reshape(H * HS, D), Wk.reshape(H * HS, D), Wv.reshape(H * HS, D)],
        axis=0)

    P = 4 if B % 4 == 0 else 1
    full = lambda shape: pl.BlockSpec(shape, lambda b: (0,) * len(shape))
    out = pl.pallas_call(
        functools.partial(_kern, heads=H, scale=D ** -0.5),
        grid=(B // P,),
        in_specs=[
            pl.BlockSpec((P, T, D_IN), lambda b: (b, 0, 0)),
            full((1, D_IN)),
            full((D, D_IN)),
            full((3 * H * HS, D)),
            full((D, H * HS)),
            full((1, D)),
            full((D, E)),
            full((E, D2, D)),
            full((E, D2)),
            full((E, D, D2)),
            full((E, D)),
        ],
        out_specs=pl.BlockSpec((P, T, D), lambda b: (b, 0, 0)),
        out_shape=jax.ShapeDtypeStruct((B, T, D), jnp.float32),
        compiler_params=pltpu.CompilerParams(
            dimension_semantics=("parallel",),
        ),
    )(x, b_wh.reshape(1, D_IN), W_wh, Wqkv, W_proj,
      b_proj.reshape(1, D), w_gate, W1, b1, W2, b2)
    return out
